# baseline (speedup 1.0000x reference)
import functools

import jax
import jax.numpy as jnp
from jax import lax
from jax.experimental import pallas as pl
from jax.experimental.pallas import tpu as pltpu
from jax.experimental.pallas import tpu_sc as plsc

_NUM_CLASSES = 21
_N = 1048576
_NW = 32
_ROWS_PER_TILE = _N // _NW
_CHUNK = 2048
_NCHUNKS = _ROWS_PER_TILE // _CHUNK
_GROUPS = _CHUNK // 16
_HIST_PAD = 448


def _cm_body(out_hbm, tgt_hbm, part_hbm, chunk_v, tgt_v, hist_v, res_v):
    wid = lax.axis_index("c") * 16 + lax.axis_index("s")
    lanes = lax.broadcasted_iota(jnp.int32, (16,), 0)
    zeros_f = jnp.zeros((16,), jnp.float32)
    ones_f = jnp.ones((16,), jnp.float32)

    def zero_body(i, carry):
        hist_v[pl.ds(i * 16, 16)] = zeros_f
        return carry

    lax.fori_loop(0, (16 * _HIST_PAD) // 16, zero_body, 0)

    def chunk_body(ci, carry):
        base = (wid * _NCHUNKS + ci) * _CHUNK
        pltpu.sync_copy(
            out_hbm.at[pl.ds(base * _NUM_CLASSES, _CHUNK * _NUM_CLASSES)],
            chunk_v)
        pltpu.sync_copy(tgt_hbm.at[pl.ds(base, _CHUNK)], tgt_v)

        def group_body(g, inner):
            row_base = (g * 16 + lanes) * _NUM_CLASSES
            best = plsc.load_gather(chunk_v, [row_base])
            arg = jnp.zeros((16,), jnp.int32)
            for c in range(1, _NUM_CLASSES):
                v = plsc.load_gather(chunk_v, [row_base + c])
                m = v > best
                best = jnp.where(m, v, best)
                arg = jnp.where(m, jnp.full((16,), c, jnp.int32), arg)
            t = tgt_v[pl.ds(g * 16, 16)]
            flat = lanes * _HIST_PAD + t * _NUM_CLASSES + arg
            plsc.addupdate_scatter(hist_v, [flat], ones_f)
            return inner

        lax.fori_loop(0, _GROUPS, group_body, 0)
        return carry

    lax.fori_loop(0, _NCHUNKS, chunk_body, 0)

    def red_body(b, carry):
        acc = zeros_f
        for l in range(16):
            acc = acc + hist_v[pl.ds(l * _HIST_PAD + b * 16, 16)]
        res_v[pl.ds(b * 16, 16)] = acc
        return carry

    lax.fori_loop(0, _HIST_PAD // 16, red_body, 0)
    pltpu.sync_copy(res_v, part_hbm.at[wid])


@jax.jit
def kernel(output, target):
    mesh = plsc.VectorSubcoreMesh(core_axis_name="c", subcore_axis_name="s")
    run = functools.partial(
        pl.kernel,
        mesh=mesh,
        out_type=jax.ShapeDtypeStruct((_NW, _HIST_PAD), jnp.float32),
        scratch_types=[
            pltpu.VMEM((_CHUNK * _NUM_CLASSES,), jnp.float32),
            pltpu.VMEM((_CHUNK,), jnp.int32),
            pltpu.VMEM((16 * _HIST_PAD,), jnp.float32),
            pltpu.VMEM((_HIST_PAD,), jnp.float32),
        ],
        compiler_params=pltpu.CompilerParams(needs_layout_passes=False),
    )(_cm_body)
    parts = run(output.reshape(-1), target)
    cm = parts.sum(axis=0)[: _NUM_CLASSES * _NUM_CLASSES]
    return cm.reshape(_NUM_CLASSES, _NUM_CLASSES)


# speedup vs baseline: 1.0882x; 1.0882x over previous
import functools

import jax
import jax.numpy as jnp
from jax import lax
from jax.experimental import pallas as pl
from jax.experimental.pallas import tpu as pltpu
from jax.experimental.pallas import tpu_sc as plsc

_NUM_CLASSES = 21
_N = 1048576
_NW = 32
_ROWS_PER_TILE = _N // _NW
_CHUNK = 2048
_NCHUNKS = _ROWS_PER_TILE // _CHUNK
_GROUPS = _CHUNK // 16
_HIST_PAD = 448


def _argmax16(chunk_ref, row_base):
    nodes = []
    for c in range(_NUM_CLASSES):
        v = plsc.load_gather(chunk_ref, [row_base + c])
        nodes.append((v, jnp.full((16,), c, jnp.int32)))
    while len(nodes) > 1:
        nxt = []
        for i in range(0, len(nodes) - 1, 2):
            (va, ia), (vb, ib) = nodes[i], nodes[i + 1]
            m = vb > va
            nxt.append((jnp.where(m, vb, va), jnp.where(m, ib, ia)))
        if len(nodes) % 2:
            nxt.append(nodes[-1])
        nodes = nxt
    return nodes[0][1]


def _cm_body(out_hbm, tgt_hbm, part_hbm, chunk0, chunk1, tgt0, tgt1,
             hist_v, res_v, sem0, sem1):
    wid = lax.axis_index("c") * 16 + lax.axis_index("s")
    lanes = lax.broadcasted_iota(jnp.int32, (16,), 0)
    zeros_f = jnp.zeros((16,), jnp.float32)
    ones_f = jnp.ones((16,), jnp.float32)
    chunks = (chunk0, chunk1)
    tgts = (tgt0, tgt1)
    sems = (sem0, sem1)

    def zero_body(i, carry):
        hist_v[pl.ds(i * 16, 16)] = zeros_f
        return carry

    lax.fori_loop(0, (16 * _HIST_PAD) // 16, zero_body, 0)

    def start(ci, buf):
        base = (wid * _NCHUNKS + ci) * _CHUNK
        a = pltpu.async_copy(
            out_hbm.at[pl.ds(base * _NUM_CLASSES, _CHUNK * _NUM_CLASSES)],
            chunks[buf], sems[buf])
        b = pltpu.async_copy(
            tgt_hbm.at[pl.ds(base, _CHUNK)], tgts[buf], sems[buf])
        return a, b

    pending = start(0, 0)
    for ci in range(_NCHUNKS):
        buf = ci % 2
        for d in pending:
            d.wait()
        if ci + 1 < _NCHUNKS:
            pending = start(ci + 1, buf ^ 1)
        chunk_v, tgt_v = chunks[buf], tgts[buf]

        def group_body(g, inner):
            row_base = (g * 16 + lanes) * _NUM_CLASSES
            arg = _argmax16(chunk_v, row_base)
            t = tgt_v[pl.ds(g * 16, 16)]
            flat = lanes * _HIST_PAD + t * _NUM_CLASSES + arg
            plsc.addupdate_scatter(hist_v, [flat], ones_f)
            return inner

        lax.fori_loop(0, _GROUPS, group_body, 0)

    def red_body(b, carry):
        acc = zeros_f
        for l in range(16):
            acc = acc + hist_v[pl.ds(l * _HIST_PAD + b * 16, 16)]
        res_v[pl.ds(b * 16, 16)] = acc
        return carry

    lax.fori_loop(0, _HIST_PAD // 16, red_body, 0)
    pltpu.sync_copy(res_v, part_hbm.at[wid])


@jax.jit
def kernel(output, target):
    mesh = plsc.VectorSubcoreMesh(core_axis_name="c", subcore_axis_name="s")
    run = functools.partial(
        pl.kernel,
        mesh=mesh,
        out_type=jax.ShapeDtypeStruct((_NW, _HIST_PAD), jnp.float32),
        scratch_types=[
            pltpu.VMEM((_CHUNK * _NUM_CLASSES,), jnp.float32),
            pltpu.VMEM((_CHUNK * _NUM_CLASSES,), jnp.float32),
            pltpu.VMEM((_CHUNK,), jnp.int32),
            pltpu.VMEM((_CHUNK,), jnp.int32),
            pltpu.VMEM((16 * _HIST_PAD,), jnp.float32),
            pltpu.VMEM((_HIST_PAD,), jnp.float32),
            pltpu.SemaphoreType.DMA,
            pltpu.SemaphoreType.DMA,
        ],
        compiler_params=pltpu.CompilerParams(needs_layout_passes=False),
    )(_cm_body)
    parts = run(output.reshape(-1), target)
    cm = parts.sum(axis=0)[: _NUM_CLASSES * _NUM_CLASSES]
    return cm.reshape(_NUM_CLASSES, _NUM_CLASSES)
